# baseline (speedup 1.0000x reference)
import functools

import jax
import jax.numpy as jnp
from jax import lax
from jax.experimental import pallas as pl
from jax.experimental.pallas import tpu as pltpu
from jax.experimental.pallas import tpu_sc as plsc

N_TRAIN = 100000
DIM = 128
K = 16
CW = 128
N_PAD = 100352
NCHUNK = N_PAD // CW
TT = 2048
BIG = 3.0e38

NC, NS = 2, 16
NW = NC * NS


_SUB = 1024


def _tc_body(x_ref, xt_ref, s_ref, cmin_ref):
    j = pl.program_id(1)
    xb = x_ref[...]
    a2 = jnp.sum(xb * xb, axis=1, keepdims=True)
    bq = xb.shape[0]
    nsc = _SUB // CW
    for u in range(TT // _SUB):
        tb = xt_ref[pl.ds(u * _SUB, _SUB), :]
        dot = lax.dot_general(xb, tb, (((1,), (1,)), ((), ())),
                              preferred_element_type=jnp.float32)
        b2 = jnp.sum(tb * tb, axis=1)[None, :]
        d2 = (a2 + b2) - 2.0 * dot
        d2 = jnp.maximum(d2, 1e-12)
        col = (j * TT + u * _SUB
               + lax.broadcasted_iota(jnp.int32, (bq, _SUB), 1))
        d2 = jnp.where(col < N_TRAIN, d2, jnp.float32(BIG))
        s_ref[:, u * _SUB:(u + 1) * _SUB] = d2
        mins = [jnp.min(d2[:, c * CW:(c + 1) * CW], axis=1)[None, :]
                for c in range(nsc)]
        cmin_ref[u * nsc:(u + 1) * nsc, :] = jnp.concatenate(mins, axis=0)


def _distances_and_chunkmins(x, x_train):
    nq = x.shape[0]
    bq = min(1024, nq)
    grid = (nq // bq, N_PAD // TT)
    return pl.pallas_call(
        _tc_body,
        grid=grid,
        in_specs=[
            pl.BlockSpec((bq, DIM), lambda i, j: (i, 0)),
            pl.BlockSpec((TT, DIM), lambda i, j: (j, 0)),
        ],
        out_specs=[
            pl.BlockSpec((bq, TT), lambda i, j: (i, j)),
            pl.BlockSpec((TT // CW, bq), lambda i, j: (j, i)),
        ],
        out_shape=[
            jax.ShapeDtypeStruct((nq, N_PAD), jnp.float32),
            jax.ShapeDtypeStruct((NCHUNK, nq), jnp.float32),
        ],
    )(x, x_train)


def _tr_body(in_ref, out_ref):
    out_ref[...] = in_ref[...].T


def _transpose_cmin(cmin_t):
    nq = cmin_t.shape[1]
    bq = min(1024, nq)
    return pl.pallas_call(
        _tr_body,
        grid=(nq // bq,),
        in_specs=[pl.BlockSpec((NCHUNK, bq), lambda i: (0, i))],
        out_specs=pl.BlockSpec((bq, NCHUNK), lambda i: (i, 0)),
        out_shape=jax.ShapeDtypeStruct((nq, NCHUNK), jnp.float32),
    )(cmin_t)


def _iota16():
    return lax.iota(jnp.int32, 16)


def _merge16(v, ids, carry):
    run_val, run_id = carry
    sv, si = plsc.sort_key_val(v, ids, descending=True)
    take = sv < run_val
    nv = jnp.where(take, sv, run_val)
    ni = jnp.where(take, si, run_id)
    return tuple(plsc.sort_key_val(nv, ni))


def _sc_select(cmin_q, s, ypad):
    nq = cmin_q.shape[0]
    qpw = nq // NW
    mesh = plsc.VectorSubcoreMesh(core_axis_name="c", subcore_axis_name="s")

    @functools.partial(
        pl.kernel,
        out_type=jax.ShapeDtypeStruct((nq,), jnp.float32),
        mesh=mesh,
        scratch_types=[
            pltpu.VMEM((2, NCHUNK), jnp.float32),
            pltpu.VMEM((K, CW), jnp.float32),
            pltpu.VMEM((N_PAD,), jnp.float32),
            pltpu.VMEM((qpw,), jnp.float32),
            pltpu.SemaphoreType.DMA,
            pltpu.SemaphoreType.DMA,
        ],
        compiler_params=pltpu.CompilerParams(needs_layout_passes=False),
    )
    def body(cmin_hbm, s_hbm, y_hbm, out_hbm, cmr, cand, yv, ob, semA, semC):
        wid = lax.axis_index("s") * NC + lax.axis_index("c")
        q0 = wid * qpw
        pltpu.async_copy(cmin_hbm.at[q0], cmr.at[0], semC)
        pltpu.sync_copy(y_hbm, yv)

        def per_query(i, acc):
            q = q0 + i
            b = i & 1
            pltpu.make_async_copy(cmin_hbm.at[q], cmr.at[b], semC).wait()
            init = (jnp.full((16,), BIG, jnp.float32),
                    jnp.zeros((16,), jnp.int32))

            def cvec(k, carry):
                return _merge16(cmr[b, pl.ds(k * 16, 16)],
                                k * 16 + _iota16(), carry)

            _, cm_id = lax.fori_loop(0, NCHUNK // 16, cvec, init)
            copies = []
            for r in range(K):
                cr = cm_id[r]
                copies.append(pltpu.async_copy(
                    s_hbm.at[q, pl.ds(cr * CW, CW)], cand.at[r], semA))

            @pl.when(i + 1 < qpw)
            def _():
                pltpu.async_copy(cmin_hbm.at[q + 1], cmr.at[1 - b], semC)

            for c in copies:
                c.wait()

            def row_body(r, carry):
                def vec(k, carry):
                    return _merge16(cand[r, pl.ds(k * 16, 16)],
                                    r * CW + k * 16 + _iota16(), carry)

                return lax.fori_loop(0, CW // 16, vec, carry)

            _, top_id = lax.fori_loop(0, K, row_body, init)
            dnums = lax.GatherDimensionNumbers(
                offset_dims=(), collapsed_slice_dims=(0,),
                start_index_map=(0,))
            gchunk = lax.gather(
                cm_id, lax.shift_right_logical(top_id, 7)[:, None], dnums,
                (1,), mode=lax.GatherScatterMode.PROMISE_IN_BOUNDS)
            gidx = gchunk * CW + lax.bitwise_and(top_id, 127)
            y16 = plsc.load_gather(yv, [gidx])
            pred = jnp.sum(y16) * jnp.float32(1.0 / K)
            return jnp.where(_iota16() == (i & 15), pred, acc)

        def qgroup(g, _):
            acc = lax.fori_loop(0, 16, lambda t, a: per_query(g * 16 + t, a),
                                jnp.zeros((16,), jnp.float32))
            ob[pl.ds(g * 16, 16)] = acc
            return 0

        lax.fori_loop(0, qpw // 16, qgroup, 0)
        pltpu.sync_copy(ob, out_hbm.at[pl.ds(q0, qpw)])

    return body(cmin_q, s, ypad)


_SPLITS = (1024, 1024, 1024, 1024)


def kernel(x, x_train, y_train, batch_size):
    del batch_size
    ypad = jnp.pad(y_train.reshape(-1), (0, N_PAD - N_TRAIN))
    preds = []
    off = 0
    for nh in _SPLITS:
        xh = lax.slice_in_dim(x, off, off + nh, axis=0)
        off += nh
        s, cmin_t = _distances_and_chunkmins(xh, x_train)
        cmin_q = _transpose_cmin(cmin_t)
        preds.append(_sc_select(cmin_q, s, ypad))
    return jnp.concatenate(preds).reshape(-1, 1)


# speedup vs baseline: 1.1366x; 1.1366x over previous
import functools

import jax
import jax.numpy as jnp
from jax import lax
from jax.experimental import pallas as pl
from jax.experimental.pallas import tpu as pltpu
from jax.experimental.pallas import tpu_sc as plsc

N_TRAIN = 100000
DIM = 128
K = 16
CW = 128
N_PAD = 100352
NCHUNK = N_PAD // CW
TT = 2048
BIG = 3.0e38

NC, NS = 2, 16
NW = NC * NS


def _tc_body(x_ref, xt_ref, s_ref, cmin_ref):
    j = pl.program_id(1)
    xb = x_ref[...]
    tb = xt_ref[...]
    dot = lax.dot_general(xb, tb, (((1,), (1,)), ((), ())),
                          preferred_element_type=jnp.float32)
    a2 = jnp.sum(xb * xb, axis=1, keepdims=True)
    b2 = jnp.sum(tb * tb, axis=1)[None, :]
    d2 = (a2 + b2) - 2.0 * dot
    d2 = jnp.maximum(d2, 1e-12)
    bq = xb.shape[0]
    col = j * TT + lax.broadcasted_iota(jnp.int32, (bq, TT), 1)
    d2 = jnp.where(col < N_TRAIN, d2, jnp.float32(BIG))
    s_ref[...] = d2
    mins = [jnp.min(d2[:, c * CW:(c + 1) * CW], axis=1)[None, :]
            for c in range(TT // CW)]
    cmin_ref[...] = jnp.concatenate(mins, axis=0)


def _distances_and_chunkmins(x, x_train):
    nq = x.shape[0]
    bq = min(1024, nq)
    grid = (nq // bq, N_PAD // TT)
    return pl.pallas_call(
        _tc_body,
        grid=grid,
        in_specs=[
            pl.BlockSpec((bq, DIM), lambda i, j: (i, 0)),
            pl.BlockSpec((TT, DIM), lambda i, j: (j, 0)),
        ],
        out_specs=[
            pl.BlockSpec((bq, TT), lambda i, j: (i, j)),
            pl.BlockSpec((TT // CW, bq), lambda i, j: (j, i)),
        ],
        out_shape=[
            jax.ShapeDtypeStruct((nq, N_PAD), jnp.float32),
            jax.ShapeDtypeStruct((NCHUNK, nq), jnp.float32),
        ],
    )(x, x_train)


def _tr_body(in_ref, out_ref):
    out_ref[...] = in_ref[...].T


def _transpose_cmin(cmin_t):
    nq = cmin_t.shape[1]
    bq = min(1024, nq)
    return pl.pallas_call(
        _tr_body,
        grid=(nq // bq,),
        in_specs=[pl.BlockSpec((NCHUNK, bq), lambda i: (0, i))],
        out_specs=pl.BlockSpec((bq, NCHUNK), lambda i: (i, 0)),
        out_shape=jax.ShapeDtypeStruct((nq, NCHUNK), jnp.float32),
    )(cmin_t)


def _iota16():
    return lax.iota(jnp.int32, 16)


def _merge16(v, ids, carry):
    run_val, run_id = carry
    sv, si = plsc.sort_key_val(v, ids, descending=True)
    take = sv < run_val
    nv = jnp.where(take, sv, run_val)
    ni = jnp.where(take, si, run_id)
    return tuple(plsc.sort_key_val(nv, ni))


def _sc_select(cmin_q, s, ypad):
    nq = cmin_q.shape[0]
    qpw = nq // NW
    mesh = plsc.VectorSubcoreMesh(core_axis_name="c", subcore_axis_name="s")

    @functools.partial(
        pl.kernel,
        out_type=jax.ShapeDtypeStruct((nq,), jnp.float32),
        mesh=mesh,
        scratch_types=[
            pltpu.VMEM((2, NCHUNK), jnp.float32),
            pltpu.VMEM((K, CW), jnp.float32),
            pltpu.VMEM((N_PAD,), jnp.float32),
            pltpu.VMEM((qpw,), jnp.float32),
            pltpu.SemaphoreType.DMA,
            pltpu.SemaphoreType.DMA,
        ],
        compiler_params=pltpu.CompilerParams(needs_layout_passes=False),
    )
    def body(cmin_hbm, s_hbm, y_hbm, out_hbm, cmr, cand, yv, ob, semA, semC):
        wid = lax.axis_index("s") * NC + lax.axis_index("c")
        q0 = wid * qpw
        pltpu.async_copy(cmin_hbm.at[q0], cmr.at[0], semC)
        pltpu.sync_copy(y_hbm, yv)

        def per_query(i, acc):
            q = q0 + i
            b = i & 1
            pltpu.make_async_copy(cmin_hbm.at[q], cmr.at[b], semC).wait()
            init = (jnp.full((16,), BIG, jnp.float32),
                    jnp.zeros((16,), jnp.int32))

            def cvec(k, carry):
                return _merge16(cmr[b, pl.ds(k * 16, 16)],
                                k * 16 + _iota16(), carry)

            _, cm_id = lax.fori_loop(0, NCHUNK // 16, cvec, init)
            copies = []
            for r in range(K):
                cr = cm_id[r]
                copies.append(pltpu.async_copy(
                    s_hbm.at[q, pl.ds(cr * CW, CW)], cand.at[r], semA))

            @pl.when(i + 1 < qpw)
            def _():
                pltpu.async_copy(cmin_hbm.at[q + 1], cmr.at[1 - b], semC)

            for c in copies:
                c.wait()

            def row_body(r, carry):
                def vec(k, carry):
                    return _merge16(cand[r, pl.ds(k * 16, 16)],
                                    r * CW + k * 16 + _iota16(), carry)

                return lax.fori_loop(0, CW // 16, vec, carry)

            _, top_id = lax.fori_loop(0, K, row_body, init)
            dnums = lax.GatherDimensionNumbers(
                offset_dims=(), collapsed_slice_dims=(0,),
                start_index_map=(0,))
            gchunk = lax.gather(
                cm_id, lax.shift_right_logical(top_id, 7)[:, None], dnums,
                (1,), mode=lax.GatherScatterMode.PROMISE_IN_BOUNDS)
            gidx = gchunk * CW + lax.bitwise_and(top_id, 127)
            y16 = plsc.load_gather(yv, [gidx])
            pred = jnp.sum(y16) * jnp.float32(1.0 / K)
            return jnp.where(_iota16() == (i & 15), pred, acc)

        def qgroup(g, _):
            acc = lax.fori_loop(0, 16, lambda t, a: per_query(g * 16 + t, a),
                                jnp.zeros((16,), jnp.float32))
            ob[pl.ds(g * 16, 16)] = acc
            return 0

        lax.fori_loop(0, qpw // 16, qgroup, 0)
        pltpu.sync_copy(ob, out_hbm.at[pl.ds(q0, qpw)])

    return body(cmin_q, s, ypad)


_SPLITS = (1024, 1024, 1024, 1024)


def kernel(x, x_train, y_train, batch_size):
    del batch_size
    ypad = jnp.pad(y_train.reshape(-1), (0, N_PAD - N_TRAIN))
    preds = []
    off = 0
    for nh in _SPLITS:
        xh = lax.slice_in_dim(x, off, off + nh, axis=0)
        off += nh
        s, cmin_t = _distances_and_chunkmins(xh, x_train)
        cmin_q = _transpose_cmin(cmin_t)
        preds.append(_sc_select(cmin_q, s, ypad))
    return jnp.concatenate(preds).reshape(-1, 1)
